# Initial kernel scaffold; baseline (speedup 1.0000x reference)
import jax
import jax.numpy as jnp
from jax.experimental import pallas as pl


def kernel(idx, emb, pos):
    raise NotImplementedError("write your pallas kernel here")


# speedup vs baseline: 1.3901x; 1.3901x over previous
import functools

import jax
import jax.numpy as jnp
from jax import lax
from jax.experimental import pallas as pl
from jax.experimental.pallas import tpu as pltpu
from jax.experimental.pallas import tpu_sc as plsc

VOCAB = 1000
SEQ = 200
D = 128
CHUNK = 40


def _emb_lookup_kernel(n_tokens, n_workers):
    rows_per_w = n_tokens // CHUNK // n_workers
    mesh = plsc.VectorSubcoreMesh(core_axis_name="c", subcore_axis_name="s")

    @functools.partial(
        pl.kernel,
        out_type=jax.ShapeDtypeStruct((n_tokens, D), jnp.float32),
        mesh=mesh,
        scratch_types=[
            pltpu.VMEM((CHUNK,), jnp.int32),
            pltpu.VMEM((CHUNK, D), jnp.float32),
            pltpu.VMEM((SEQ, D), jnp.float32),
            pltpu.SemaphoreType.DMA,
        ],
    )
    def body(idx_hbm, emb_hbm, pos_hbm, out_hbm, idx_v, rows_v, pos_v, sem):
        nc = lax.axis_size("c")
        wid = lax.axis_index("s") * nc + lax.axis_index("c")

        pltpu.sync_copy(pos_hbm.at[pl.ds(0, SEQ)], pos_v)

        r0 = wid * rows_per_w

        def chunk_body(r, _):
            pltpu.sync_copy(idx_hbm.at[r], idx_v)
            pltpu.async_copy(emb_hbm.at[idx_v], rows_v, sem).wait()
            poff = (r % (SEQ // CHUNK)) * CHUNK

            def add_body(i, _):
                p = poff + i
                for j in range(D // 16):
                    sl = pl.ds(j * 16, 16)
                    rows_v[i, sl] = rows_v[i, sl] + pos_v[p, sl]
                return 0

            lax.fori_loop(0, CHUNK, add_body, 0)
            pltpu.sync_copy(rows_v, out_hbm.at[pl.ds(r * CHUNK, CHUNK)])
            return 0

        lax.fori_loop(r0, r0 + rows_per_w, chunk_body, 0)

    return body


def kernel(idx, emb, pos):
    b, l = idx.shape
    n_tokens = b * l
    info = plsc.get_sparse_core_info()
    n_workers = info.num_cores * info.num_subcores
    idx2 = idx.reshape(n_tokens // CHUNK, CHUNK).astype(jnp.int32)
    out = _emb_lookup_kernel(n_tokens, n_workers)(idx2, emb, pos)
    return out.reshape(b, l, D)


# speedup vs baseline: 2.4130x; 1.7359x over previous
import functools

import jax
import jax.numpy as jnp
from jax import lax
from jax.experimental import pallas as pl
from jax.experimental.pallas import tpu as pltpu
from jax.experimental.pallas import tpu_sc as plsc

SEQ = 200
D = 128
CHUNK = 128
POS_EXT = SEQ + CHUNK


def _emb_lookup_kernel(n_tokens, n_workers):
    chunks_per_w = n_tokens // CHUNK // n_workers
    pairs = chunks_per_w // 2
    mesh = plsc.VectorSubcoreMesh(core_axis_name="c", subcore_axis_name="s")

    @functools.partial(
        pl.kernel,
        out_type=jax.ShapeDtypeStruct((n_tokens, D), jnp.float32),
        mesh=mesh,
        scratch_types=[
            pltpu.VMEM((chunks_per_w, CHUNK), jnp.int32),
            pltpu.VMEM((CHUNK, D), jnp.float32),
            pltpu.VMEM((CHUNK, D), jnp.float32),
            pltpu.VMEM((POS_EXT, D), jnp.float32),
            pltpu.SemaphoreType.DMA,
            pltpu.SemaphoreType.DMA,
            pltpu.SemaphoreType.DMA,
            pltpu.SemaphoreType.DMA,
        ],
    )
    def body(idx_hbm, emb_hbm, pos_hbm, out_hbm, idx_v, rows_a, rows_b,
             pos_v, gsem_a, gsem_b, osem_a, osem_b):
        nc = lax.axis_size("c")
        wid = lax.axis_index("s") * nc + lax.axis_index("c")
        c0 = wid * chunks_per_w

        pltpu.sync_copy(pos_hbm.at[pl.ds(0, SEQ)], pos_v.at[pl.ds(0, SEQ)])
        pltpu.sync_copy(pos_hbm.at[pl.ds(0, CHUNK)],
                        pos_v.at[pl.ds(SEQ, CHUNK)])
        pltpu.sync_copy(idx_hbm.at[pl.ds(c0, chunks_per_w)], idx_v)

        def gather(g_local, rows, sem):
            return pltpu.async_copy(emb_hbm.at[idx_v.at[g_local]], rows, sem)

        def gather_wait(g_local, rows, sem):
            pltpu.make_async_copy(emb_hbm.at[idx_v.at[g_local]], rows,
                                  sem).wait()

        def out_slice(g_local):
            return out_hbm.at[pl.ds((c0 + g_local) * CHUNK, CHUNK)]

        def add_pos(rows, poff):
            def body_i(i, _):
                p = poff + i
                for j in range(D // 16):
                    sl = pl.ds(j * 16, 16)
                    rows[i, sl] = rows[i, sl] + pos_v[p, sl]
                return 0

            lax.fori_loop(0, CHUNK, body_i, 0)

        gather(0, rows_a, gsem_a)

        def pair_body(t, _):
            g0 = 2 * t
            g1 = g0 + 1
            @pl.when(t > 0)
            def _():
                pltpu.make_async_copy(rows_b, out_slice(g1 - 2), osem_b).wait()

            gather(g1, rows_b, gsem_b)

            gather_wait(g0, rows_a, gsem_a)
            add_pos(rows_a, lax.rem(g0 * CHUNK, SEQ))
            pltpu.async_copy(rows_a, out_slice(g0), osem_a)

            gather_wait(g1, rows_b, gsem_b)
            pltpu.make_async_copy(rows_a, out_slice(g0), osem_a).wait()

            @pl.when(t + 1 < pairs)
            def _():
                gather(g0 + 2, rows_a, gsem_a)

            add_pos(rows_b, lax.rem(g1 * CHUNK, SEQ))
            pltpu.async_copy(rows_b, out_slice(g1), osem_b)
            return 0

        lax.fori_loop(0, pairs, pair_body, 0)
        pltpu.make_async_copy(rows_b, out_slice(chunks_per_w - 1),
                              osem_b).wait()

    return body


def kernel(idx, emb, pos):
    b, l = idx.shape
    n_tokens = b * l
    info = plsc.get_sparse_core_info()
    n_workers = info.num_cores * info.num_subcores
    idx2 = idx.reshape(n_tokens // CHUNK, CHUNK).astype(jnp.int32)
    out = _emb_lookup_kernel(n_tokens, n_workers)(idx2, emb, pos)
    return out.reshape(b, l, D)


# speedup vs baseline: 6.0582x; 2.5106x over previous
import functools

import jax
import jax.numpy as jnp
from jax import lax
from jax.experimental import pallas as pl
from jax.experimental.pallas import tpu as pltpu
from jax.experimental.pallas import tpu_sc as plsc

SEQ = 200
D = 128
CHUNK = 128
POS_EXT = SEQ + CHUNK


def _emb_lookup_kernel(n_tokens, n_workers):
    chunks_per_w = n_tokens // CHUNK // n_workers
    pairs = chunks_per_w // 2
    mesh = plsc.VectorSubcoreMesh(core_axis_name="c", subcore_axis_name="s")

    @functools.partial(
        pl.kernel,
        out_type=jax.ShapeDtypeStruct((n_tokens, D), jnp.float32),
        mesh=mesh,
        scratch_types=[
            pltpu.VMEM((chunks_per_w, CHUNK), jnp.int32),
            pltpu.VMEM((CHUNK, D), jnp.float32),
            pltpu.VMEM((CHUNK, D), jnp.float32),
            pltpu.VMEM((POS_EXT, D), jnp.float32),
            pltpu.SemaphoreType.DMA,
            pltpu.SemaphoreType.DMA,
            pltpu.SemaphoreType.DMA,
            pltpu.SemaphoreType.DMA,
        ],
    )
    def body(idx_hbm, emb_hbm, pos_hbm, out_hbm, idx_v, rows_a, rows_b,
             pos_v, gsem_a, gsem_b, osem_a, osem_b):
        nc = lax.axis_size("c")
        wid = lax.axis_index("s") * nc + lax.axis_index("c")
        c0 = wid * chunks_per_w

        pltpu.sync_copy(pos_hbm.at[pl.ds(0, SEQ)], pos_v.at[pl.ds(0, SEQ)])
        pltpu.sync_copy(pos_hbm.at[pl.ds(0, CHUNK)],
                        pos_v.at[pl.ds(SEQ, CHUNK)])
        pltpu.sync_copy(idx_hbm.at[pl.ds(c0, chunks_per_w)], idx_v)

        def gather(g_local, rows, sem):
            return pltpu.async_copy(emb_hbm.at[idx_v.at[g_local]], rows, sem)

        def gather_wait(g_local, rows, sem):
            pltpu.make_async_copy(emb_hbm.at[idx_v.at[g_local]], rows,
                                  sem).wait()

        def out_slice(g_local):
            return out_hbm.at[pl.ds((c0 + g_local) * CHUNK, CHUNK)]

        def add_pos(rows, poff):
            @plsc.parallel_loop(0, CHUNK, step=1, unroll=4)
            def _(i):
                p = poff + i
                for j in range(D // 16):
                    sl = pl.ds(j * 16, 16)
                    rows[i, sl] = rows[i, sl] + pos_v[p, sl]

        gather(0, rows_a, gsem_a)

        def pair_body(t, _):
            g0 = 2 * t
            g1 = g0 + 1
            @pl.when(t > 0)
            def _():
                pltpu.make_async_copy(rows_b, out_slice(g1 - 2), osem_b).wait()

            gather(g1, rows_b, gsem_b)

            gather_wait(g0, rows_a, gsem_a)
            add_pos(rows_a, lax.rem(g0 * CHUNK, SEQ))
            pltpu.async_copy(rows_a, out_slice(g0), osem_a)

            gather_wait(g1, rows_b, gsem_b)
            pltpu.make_async_copy(rows_a, out_slice(g0), osem_a).wait()

            @pl.when(t + 1 < pairs)
            def _():
                gather(g0 + 2, rows_a, gsem_a)

            add_pos(rows_b, lax.rem(g1 * CHUNK, SEQ))
            pltpu.async_copy(rows_b, out_slice(g1), osem_b)
            return 0

        lax.fori_loop(0, pairs, pair_body, 0)
        pltpu.make_async_copy(rows_b, out_slice(chunks_per_w - 1),
                              osem_b).wait()

    return body


def kernel(idx, emb, pos):
    b, l = idx.shape
    n_tokens = b * l
    info = plsc.get_sparse_core_info()
    n_workers = info.num_cores * info.num_subcores
    idx2 = idx.reshape(n_tokens // CHUNK, CHUNK).astype(jnp.int32)
    out = _emb_lookup_kernel(n_tokens, n_workers)(idx2, emb, pos)
    return out.reshape(b, l, D)
